# baseline (speedup 1.0000x reference)
import functools

import jax
import jax.numpy as jnp
from jax import lax
from jax.experimental import pallas as pl
from jax.experimental.pallas import tpu as pltpu
from jax.experimental.pallas import tpu_sc as plsc

_R = 256
_C = 512
_LANES = 128
_G = 128


def _argmin_body(nchunks, xb_ref, x2_ref, w_ref, cb2_ref, idx_ref):
    xb = xb_ref[...]
    x2 = x2_ref[...]
    r = xb.shape[1]

    lane = lax.broadcasted_iota(jnp.int32, (r, _LANES), 1)

    def chunk(c, carry):
        acc, argidx = carry
        w = w_ref[pl.ds(c * _C, _C), :]
        cb2 = cb2_ref[:, pl.ds(c * _C, _C)]
        dott = lax.dot_general(
            w, xb, (((1,), (0,)), ((), ())),
            preferred_element_type=jnp.float32)
        dot = dott.T
        dist = (x2 - 2.0 * dot) + cb2
        for j in range(_C // _LANES):
            d = dist[:, j * _LANES:(j + 1) * _LANES]
            better = d < acc
            acc = jnp.where(better, d, acc)
            argidx = jnp.where(better, c * _C + j * _LANES + lane, argidx)
        return acc, argidx

    acc0 = jnp.full((r, _LANES), jnp.inf, dtype=jnp.float32)
    ai0 = jnp.zeros((r, _LANES), dtype=jnp.int32)
    acc, argidx = lax.fori_loop(0, nchunks, chunk, (acc0, ai0))

    m = jnp.min(acc, axis=1, keepdims=True)
    cand = jnp.where(acc == m, argidx, jnp.full_like(argidx, 2**30))
    idx_ref[...] = jnp.min(cand, axis=1, keepdims=True)


def _argmin_indices(flat_x, codebook, x2, cb2, *, interpret=False):
    n, d = flat_x.shape
    v = codebook.shape[0]
    kp = 128
    xbt = jnp.pad(flat_x.astype(jnp.bfloat16).T, ((0, kp - d), (0, 0)))
    w2 = jnp.pad(codebook, ((0, 0), (0, kp - d)))
    return pl.pallas_call(
        functools.partial(_argmin_body, v // _C),
        grid=(n // _R,),
        in_specs=[
            pl.BlockSpec((kp, _R), lambda i: (0, i)),
            pl.BlockSpec((_R, 1), lambda i: (i, 0)),
            pl.BlockSpec((v, kp), lambda i: (0, 0)),
            pl.BlockSpec((1, v), lambda i: (0, 0)),
        ],
        out_specs=pl.BlockSpec((_R, 1), lambda i: (i, 0)),
        out_shape=jax.ShapeDtypeStruct((n, 1), jnp.int32),
        interpret=interpret,
    )(xbt, x2, w2, cb2)


def _sc_gather(codebook, idx2d):
    nrow, d = idx2d.shape[0], codebook.shape[1]
    info = plsc.get_sparse_core_info()
    nw = info.num_cores * info.num_subcores
    rows_per_w = nrow // nw

    mesh = plsc.VectorSubcoreMesh(core_axis_name="c", subcore_axis_name="s")

    @functools.partial(
        pl.kernel,
        mesh=mesh,
        out_type=jax.ShapeDtypeStruct((nrow, _G, d), jnp.float32),
        scratch_types=[
            pltpu.VMEM((rows_per_w, _G), jnp.int32),
            pltpu.VMEM((rows_per_w, _G, d), jnp.float32),
            pltpu.SemaphoreType.DMA,
        ],
        compiler_params=pltpu.CompilerParams(use_tc_tiling_on_sc=False),
    )
    def gather_kernel(cb_hbm, idx_hbm, out_hbm, idx_v, rows_v, sem):
        wid = lax.axis_index("s") * info.num_cores + lax.axis_index("c")
        base = wid * rows_per_w
        pltpu.sync_copy(idx_hbm.at[pl.ds(base, rows_per_w)], idx_v)
        copies = [
            pltpu.async_copy(cb_hbm.at[idx_v.at[j]], rows_v.at[j], sem)
            for j in range(rows_per_w)
        ]
        for cpy in copies:
            cpy.wait()
        pltpu.sync_copy(rows_v, out_hbm.at[pl.ds(base, rows_per_w)])

    return gather_kernel(codebook, idx2d)


def kernel(x, codebook):
    d = codebook.shape[1]
    flat_x = x.reshape(-1, d)
    x2 = jnp.sum(flat_x ** 2, axis=1, keepdims=True)
    cb2 = jnp.sum(codebook ** 2, axis=1)[None, :]
    idx = _argmin_indices(flat_x, codebook, x2, cb2)
    idx2d = idx.reshape(-1, _G)
    out = _sc_gather(codebook, idx2d)
    return out.reshape(x.shape)
